# baseline (speedup 1.0000x reference)
import functools

import jax
import jax.numpy as jnp
from jax import lax
from jax.experimental import pallas as pl
from jax.experimental.pallas import tpu as pltpu
from jax.experimental.pallas import tpu_sc as plsc

_B = 256
_D = 128
_K = 4096
_T = 0.07
_INV_T = 1.0 / _T

_NW = 32
_QW = _B // _NW
_CH = 128
_NCH = _K // _CH
_DV = _D // 16
_LN2 = 0.6931471805599453


def _rsqrt16(n2):
    iv = plsc.bitcast(n2, jnp.int32)
    y = plsc.bitcast(jnp.int32(0x5F3759DF) - (iv >> 1), jnp.float32)
    for _ in range(3):
        y = y * (1.5 - 0.5 * n2 * y * y)
    return y


def _ln16(x):
    bits = plsc.bitcast(x, jnp.int32)
    e = ((bits >> 23) & 0xFF) - 127
    f = plsc.bitcast((bits & 0x007FFFFF) | jnp.int32(0x3F800000),
                     jnp.float32)
    z = (f - 1.0) / (f + 1.0)
    z2 = z * z
    lnf = z * (2.0 + z2 * (2.0 / 3.0 + z2 * (2.0 / 5.0 + z2 * (2.0 / 7.0))))
    return e.astype(jnp.float32) * _LN2 + lnf


def _sc_body(fq_hbm, fk_hbm, nidx_hbm, bank_hbm, fkn_hbm, part_hbm,
             fq_v, fk_v, fkn_v, idx_v, bufs, log_v, mx_v, la_v, red_v,
             out_v, shared, sems):
    cid = lax.axis_index("c")
    sid = lax.axis_index("s")
    wid = sid * 2 + cid
    qbase = wid * _QW
    pltpu.sync_copy(fq_hbm.at[pl.ds(qbase, _QW)], fq_v)
    pltpu.sync_copy(fk_hbm.at[pl.ds(qbase, _QW)], fk_v)
    pltpu.sync_copy(nidx_hbm.at[wid, 0], idx_v.at[0])
    lanes = lax.iota(jnp.int32, 16)
    zero16 = jnp.zeros((16,), jnp.float32)
    neg16 = jnp.full((16,), -jnp.inf, jnp.float32)
    la_v[...] = zero16

    def _compute_chunk(qq, cc, buf):
        fqv = [fq_v[qq, pl.ds(16 * j, 16)] for j in range(_DV)]

        @pl.loop(0, _CH // 16)
        def _group(g):
            acc = zero16
            for i in range(16):
                r = g * 16 + i
                v = buf[r, pl.ds(0, 16)] * fqv[0]
                for j in range(1, _DV):
                    v = v + buf[r, pl.ds(16 * j, 16)] * fqv[j]
                acc = jnp.where(lanes == i, jnp.sum(v), acc)
            log_v[pl.ds(cc * _CH + g * 16, 16)] = acc
            mx_v[...] = jnp.maximum(mx_v[...], acc)

    def _start(t, k):
        qq, cc = t // _NCH, t % _NCH
        for h in range(2):
            pltpu.async_copy(
                bank_hbm.at[idx_v.at[qq, cc, pl.ds(64 * h, 64)]],
                bufs.at[k, pl.ds(64 * h, 64)], sems.at[2 * k + h])

    def _wait(t, k):
        qq, cc = t // _NCH, t % _NCH
        for h in range(2):
            pltpu.make_async_copy(
                bank_hbm.at[idx_v.at[qq, cc, pl.ds(64 * h, 64)]],
                bufs.at[k, pl.ds(64 * h, 64)], sems.at[2 * k + h]).wait()

    _NT = _QW * _NCH
    for k in range(3):
        _start(k, k)
    pltpu.sync_copy(nidx_hbm.at[wid, pl.ds(1, _QW - 1)],
                    idx_v.at[pl.ds(1, _QW - 1)])

    @pl.loop(0, _NT, step=4)
    def _chunk(t0):
        for k in range(4):
            t = t0 + k
            qq = t // _NCH
            cc = t % _NCH

            @pl.when(cc == 0)
            def _():
                mx_v[...] = neg16

            @pl.when(t + 3 < _NT)
            def _():
                _start(t + 3, (k + 3) % 4)

            _wait(t, k)
            _compute_chunk(qq, cc, bufs.at[k])

            @pl.when(cc == _NCH - 1)
            def _():
                fqv = [fq_v[qq, pl.ds(16 * j, 16)] for j in range(_DV)]
                nv = fqv[0] * fqv[0]
                for j in range(1, _DV):
                    nv = nv + fqv[j] * fqv[j]
                nq2 = jnp.full((16,), jnp.maximum(jnp.sum(nv), 1e-24),
                               jnp.float32)
                sc = (_rsqrt16(nq2) * _INV_T)[0]
                m_q = jnp.max(mx_v[...]) * sc
                svec = pl.loop(0, _K // 16, init_carry=zero16)(
                    lambda j, s:
                        s + jnp.exp(log_v[pl.ds(16 * j, 16)] * sc - m_q))
                s_q = jnp.sum(svec)

                fkv = [fk_v[qq, pl.ds(16 * j, 16)] for j in range(_DV)]
                kv = fkv[0] * fkv[0]
                dv = fqv[0] * fkv[0]
                for j in range(1, _DV):
                    kv = kv + fkv[j] * fkv[j]
                    dv = dv + fqv[j] * fkv[j]
                nk2 = jnp.full((16,), jnp.maximum(jnp.sum(kv), 1e-24),
                               jnp.float32)
                yk = _rsqrt16(nk2)
                for j in range(_DV):
                    fkn_v[qq, pl.ds(16 * j, 16)] = fkv[j] * yk
                dl_q = jnp.sum(dv) * sc * yk[0]

                mqv = jnp.full((16,), m_q, jnp.float32)
                dlv = jnp.full((16,), dl_q, jnp.float32)
                sqv = jnp.full((16,), s_q, jnp.float32)
                mt = jnp.maximum(mqv, dlv)
                st = sqv * jnp.exp(mqv - mt) + jnp.exp(dlv - mt)
                contrib = (_ln16(st) + mt - dlv) * (1.0 / _B)
                la_v[...] = la_v[...] + jnp.where(lanes == qq, contrib, 0.0)

    pltpu.sync_copy(fkn_v, fkn_hbm.at[pl.ds(qbase, _QW)])
    pltpu.sync_copy(la_v, shared.at[sid])
    plsc.subcore_barrier()

    @pl.when(sid == 0)
    def _():
        pltpu.sync_copy(shared, red_v)
        tot = pl.loop(0, 16, init_carry=zero16)(
            lambda j, a: a + red_v[j, pl.ds(0, 16)])
        out_v[...] = jnp.where(lanes == 0, jnp.sum(tot), 0.0)
        pltpu.sync_copy(out_v, part_hbm.at[cid])


_sc_kernel = functools.partial(
    pl.kernel,
    out_type=(jax.ShapeDtypeStruct((_B, _D), jnp.float32),
              jax.ShapeDtypeStruct((2, 16), jnp.float32)),
    mesh=plsc.VectorSubcoreMesh(core_axis_name="c", subcore_axis_name="s"),
    compiler_params=pltpu.CompilerParams(needs_layout_passes=False),
    scratch_types=[
        pltpu.VMEM((_QW, _D), jnp.float32),
        pltpu.VMEM((_QW, _D), jnp.float32),
        pltpu.VMEM((_QW, _D), jnp.float32),
        pltpu.VMEM((_QW, _NCH, _CH), jnp.int32),
        pltpu.VMEM((4, _CH, _D), jnp.float32),
        pltpu.VMEM((_K,), jnp.float32),
        pltpu.VMEM((16,), jnp.float32),
        pltpu.VMEM((16,), jnp.float32),
        pltpu.VMEM((16, 16), jnp.float32),
        pltpu.VMEM((16,), jnp.float32),
        pltpu.VMEM_SHARED((16, 16), jnp.float32),
        pltpu.SemaphoreType.DMA((8,)),
    ],
)(_sc_body)


def kernel(idx, fea_q, fea_k, gpu_idx, bank, noise_idx):
    del idx, gpu_idx
    nidx4 = noise_idx.reshape(_NW, _QW, _NCH, _CH)
    fkn, part = _sc_kernel(fea_q, fea_k, nidx4, bank)
    loss = part[0, 0] + part[1, 0]
    return (loss, fkn)
